# baseline (speedup 1.0000x reference)
import functools

import jax
import jax.numpy as jnp
from jax import lax
from jax.experimental import pallas as pl
from jax.experimental.pallas import tpu as pltpu
from jax.experimental.pallas import tpu_sc as plsc

NUM_ROWS = 15
DIM = 32

_mesh = plsc.VectorSubcoreMesh(core_axis_name="c", subcore_axis_name="s")


@functools.partial(
    pl.kernel,
    out_type=jax.ShapeDtypeStruct((1, DIM), jnp.float32),
    mesh=_mesh,
    scratch_types=[
        pltpu.VMEM((16,), jnp.int32),
        pltpu.VMEM((1, DIM), jnp.float32),
    ],
)
def _lookup(table_hbm, id_hbm, out_hbm, idx_v, row_v):
    c = lax.axis_index("c")
    s = lax.axis_index("s")

    @pl.when(jnp.logical_and(c == 0, s == 0))
    def _():
        pltpu.sync_copy(id_hbm, idx_v.at[pl.ds(0, 1)])
        i = idx_v[...][0]
        pltpu.sync_copy(table_hbm.at[pl.ds(i, 1)], row_v)
        pltpu.sync_copy(row_v, out_hbm)


def kernel(table, decision_id):
    out = _lookup(table, decision_id.reshape(1).astype(jnp.int32))
    return out.reshape(DIM)


# speedup vs baseline: 1.1584x; 1.1584x over previous
import functools

import jax
import jax.numpy as jnp
from jax import lax
from jax.experimental import pallas as pl
from jax.experimental.pallas import tpu as pltpu
from jax.experimental.pallas import tpu_sc as plsc

NUM_ROWS = 15
DIM = 32

_mesh = plsc.ScalarSubcoreMesh(axis_name="c", num_cores=1)


@functools.partial(
    pl.kernel,
    out_type=jax.ShapeDtypeStruct((1, DIM), jnp.float32),
    mesh=_mesh,
    scratch_types=[
        pltpu.SMEM((1,), jnp.int32),
        pltpu.SMEM((1, DIM), jnp.float32),
    ],
)
def _lookup(table_hbm, id_hbm, out_hbm, id_s, row_s):
    pltpu.sync_copy(id_hbm, id_s)
    i = id_s[0]
    pltpu.sync_copy(table_hbm.at[pl.ds(i, 1)], row_s)
    pltpu.sync_copy(row_s, out_hbm)


def kernel(table, decision_id):
    out = _lookup(table, decision_id.reshape(1).astype(jnp.int32))
    return out.reshape(DIM)


# speedup vs baseline: 1.1700x; 1.0101x over previous
import functools

import jax
import jax.numpy as jnp
from jax import lax
from jax.experimental import pallas as pl
from jax.experimental.pallas import tpu as pltpu
from jax.experimental.pallas import tpu_sc as plsc

NUM_ROWS = 15
DIM = 32

_mesh = plsc.ScalarSubcoreMesh(axis_name="c", num_cores=1)


@functools.partial(
    pl.kernel,
    out_type=jax.ShapeDtypeStruct((1, DIM), jnp.float32),
    mesh=_mesh,
    scratch_types=[
        pltpu.SMEM((1,), jnp.int32),
    ],
)
def _lookup(table_hbm, id_hbm, out_hbm, id_s):
    pltpu.sync_copy(id_hbm, id_s)
    i = id_s[0]
    pltpu.sync_copy(table_hbm.at[pl.ds(i, 1)], out_hbm)


def kernel(table, decision_id):
    out = _lookup(table, decision_id.reshape(1).astype(jnp.int32))
    return out.reshape(DIM)


# speedup vs baseline: 6.2190x; 5.3152x over previous
import jax
import jax.numpy as jnp
from jax.experimental import pallas as pl
from jax.experimental.pallas import tpu as pltpu

NUM_ROWS = 15
DIM = 32


def _body(idx_ref, row_ref, out_ref):
    out_ref[...] = row_ref[0]


def kernel(table, decision_id):
    out = pl.pallas_call(
        _body,
        grid_spec=pltpu.PrefetchScalarGridSpec(
            num_scalar_prefetch=1,
            grid=(1,),
            in_specs=[
                pl.BlockSpec((1, 1, DIM), lambda i, idx_ref: (idx_ref[0], 0, 0))
            ],
            out_specs=pl.BlockSpec((1, DIM), lambda i, idx_ref: (0, 0)),
        ),
        out_shape=jax.ShapeDtypeStruct((1, DIM), jnp.float32),
    )(decision_id.reshape(1), table.reshape(NUM_ROWS, 1, DIM))
    return out.reshape(DIM)
